# Initial kernel scaffold; baseline (speedup 1.0000x reference)
import jax
import jax.numpy as jnp
from jax.experimental import pallas as pl


def kernel(x_item, x_seq, edge_index, W_item, W_seq, lin0, att_src0, att_dst0, bias0, lin1, att_src1, att_dst1, bias1):
    raise NotImplementedError("write your pallas kernel here")


# speedup vs baseline: 19.9796x; 19.9796x over previous
import functools

import jax
import jax.numpy as jnp
from jax import lax
from jax.experimental import pallas as pl
from jax.experimental.pallas import tpu as pltpu
from jax.experimental.pallas import tpu_sc as plsc

N_ITEM = 5000
N_SEQ = 5000
N = N_ITEM + N_SEQ
E = 160000
D = 128
NH = 4
BN_EPS = 1e-5

NPAD = 10112
E_EXT = E + N
T_EDGE = 5376
EPAD = 32 * T_EDGE


def _mm_kernel(a_ref, b_ref, o_ref):
    o_ref[...] = jnp.dot(a_ref[...], b_ref[...],
                         preferred_element_type=jnp.float32,
                         precision=lax.Precision.DEFAULT)


def _mm_acc_kernel(a_ref, b_ref, o_ref):
    k = pl.program_id(1)

    @pl.when(k == 0)
    def _():
        o_ref[...] = jnp.zeros_like(o_ref)

    o_ref[...] += jnp.dot(a_ref[...], b_ref[...],
                          preferred_element_type=jnp.float32,
                          precision=lax.Precision.DEFAULT)


def _h_item(x_item, W_itemT):
    return pl.pallas_call(
        _mm_kernel,
        grid=(1,),
        in_specs=[pl.BlockSpec((N_ITEM, D), lambda i: (0, 0)),
                  pl.BlockSpec((D, D), lambda i: (0, 0))],
        out_specs=pl.BlockSpec((N_ITEM, D), lambda i: (0, 0)),
        out_shape=jax.ShapeDtypeStruct((N_ITEM, D), jnp.float32),
    )(x_item, W_itemT)


def _h_seq(x_seq, W_seqT):
    MB = 1000
    return pl.pallas_call(
        _mm_kernel,
        grid=(N_SEQ // MB,),
        in_specs=[pl.BlockSpec((MB, N_ITEM), lambda i: (i, 0)),
                  pl.BlockSpec((N_ITEM, D), lambda i: (0, 0))],
        out_specs=pl.BlockSpec((MB, D), lambda i: (i, 0)),
        out_shape=jax.ShapeDtypeStruct((N_SEQ, D), jnp.float32),
    )(x_seq, W_seqT)


def _proj_kernel(h_ref, linT_ref, ab_ref, x_ref, a_ref):
    xb = jnp.dot(h_ref[...], linT_ref[...],
                 preferred_element_type=jnp.float32,
                 precision=lax.Precision.DEFAULT)
    x_ref[...] = xb
    a_ref[...] = jnp.dot(xb, ab_ref[...],
                         preferred_element_type=jnp.float32,
                         precision=lax.Precision.DEFAULT)


def _proj(H, linT, AB):
    MB = 2000
    return pl.pallas_call(
        _proj_kernel,
        grid=(N // MB,),
        in_specs=[pl.BlockSpec((MB, D), lambda i: (i, 0)),
                  pl.BlockSpec((D, NH * D), lambda i: (0, 0)),
                  pl.BlockSpec((NH * D, D), lambda i: (0, 0))],
        out_specs=[pl.BlockSpec((MB, NH * D), lambda i: (i, 0)),
                   pl.BlockSpec((MB, D), lambda i: (i, 0))],
        out_shape=[jax.ShapeDtypeStruct((N, NH * D), jnp.float32),
                   jax.ShapeDtypeStruct((N, D), jnp.float32)],
    )(H, linT, AB)


def _stats_kernel(g0_ref, g1_ref, b_ref, r_ref, s1_ref, s2_ref):
    i = pl.program_id(0)
    rb = jnp.maximum(g0_ref[...] + g1_ref[...] + b_ref[...], 0.0)
    r_ref[...] = rb

    @pl.when(i == 0)
    def _():
        s1_ref[...] = jnp.zeros_like(s1_ref)
        s2_ref[...] = jnp.zeros_like(s2_ref)

    s1_ref[...] += jnp.sum(rb, axis=0, keepdims=True)
    s2_ref[...] += jnp.sum(rb * rb, axis=0, keepdims=True)


def _stats(g0, g1, bias):
    MB = 2000
    return pl.pallas_call(
        _stats_kernel,
        grid=(N // MB,),
        in_specs=[pl.BlockSpec((MB, D), lambda i: (i, 0)),
                  pl.BlockSpec((MB, D), lambda i: (i, 0)),
                  pl.BlockSpec((1, D), lambda i: (0, 0))],
        out_specs=[pl.BlockSpec((MB, D), lambda i: (i, 0)),
                   pl.BlockSpec((1, D), lambda i: (0, 0)),
                   pl.BlockSpec((1, D), lambda i: (0, 0))],
        out_shape=[jax.ShapeDtypeStruct((N, D), jnp.float32),
                   jax.ShapeDtypeStruct((1, D), jnp.float32),
                   jax.ShapeDtypeStruct((1, D), jnp.float32)],
    )(g0, g1, bias)


def _norm_kernel(r_ref, s1_ref, s2_ref, o_ref):
    m = s1_ref[...] / N
    v = s2_ref[...] / N - m * m
    o_ref[...] = (r_ref[...] - m) * lax.rsqrt(v + BN_EPS)


def _norm(R, s1, s2):
    MB = 2000
    return pl.pallas_call(
        _norm_kernel,
        grid=(N // MB,),
        in_specs=[pl.BlockSpec((MB, D), lambda i: (i, 0)),
                  pl.BlockSpec((1, D), lambda i: (0, 0)),
                  pl.BlockSpec((1, D), lambda i: (0, 0))],
        out_specs=pl.BlockSpec((MB, D), lambda i: (i, 0)),
        out_shape=jax.ShapeDtypeStruct((N, D), jnp.float32),
    )(R, s1, s2)


def _normproj_kernel(r_ref, s1_ref, s2_ref, linT_ref, ab_ref, x_ref, a_ref):
    m = s1_ref[...] / N
    v = s2_ref[...] / N - m * m
    hn = (r_ref[...] - m) * lax.rsqrt(v + BN_EPS)
    xb = jnp.dot(hn, linT_ref[...],
                 preferred_element_type=jnp.float32,
                 precision=lax.Precision.DEFAULT)
    x_ref[...] = xb
    a_ref[...] = jnp.dot(xb, ab_ref[...],
                         preferred_element_type=jnp.float32,
                         precision=lax.Precision.DEFAULT)


def _normproj(R, s1, s2, linT, AB):
    MB = 2000
    return pl.pallas_call(
        _normproj_kernel,
        grid=(N // MB,),
        in_specs=[pl.BlockSpec((MB, D), lambda i: (i, 0)),
                  pl.BlockSpec((1, D), lambda i: (0, 0)),
                  pl.BlockSpec((1, D), lambda i: (0, 0)),
                  pl.BlockSpec((D, NH * D), lambda i: (0, 0)),
                  pl.BlockSpec((NH * D, D), lambda i: (0, 0))],
        out_specs=[pl.BlockSpec((MB, NH * D), lambda i: (i, 0)),
                   pl.BlockSpec((MB, D), lambda i: (i, 0))],
        out_shape=[jax.ShapeDtypeStruct((N, NH * D), jnp.float32),
                   jax.ShapeDtypeStruct((N, D), jnp.float32)],
    )(R, s1, s2, linT, AB)


def _winv_kernel(d0_ref, d1_ref, o_ref):
    o_ref[...] = 0.25 / (d0_ref[...] + d1_ref[...] + 1e-16)


def _winv(d0, d1):
    R = NPAD * 16 // 128
    return pl.pallas_call(
        _winv_kernel,
        grid=(1,),
        in_specs=[pl.BlockSpec((R, 128), lambda i: (0, 0)),
                  pl.BlockSpec((R, 128), lambda i: (0, 0))],
        out_specs=pl.BlockSpec((R, 128), lambda i: (0, 0)),
        out_shape=jax.ShapeDtypeStruct((R, 128), jnp.float32),
    )(d0, d1)


NC = 2
NS = 16
C1 = 128
C2 = 64
ROWS_T = NPAD // NS
_I16 = lambda: jnp.arange(16, dtype=jnp.int32)


def _sc_pass1(s_ext, d_ext, asrc_flat, adst_flat):
    mesh = plsc.VectorSubcoreMesh(core_axis_name="c", subcore_axis_name="s", num_cores=NC, num_subcores=NS)

    @functools.partial(
        pl.kernel,
        out_type=(jax.ShapeDtypeStruct((NC * NPAD, 16), jnp.float32),
                  jax.ShapeDtypeStruct((EPAD * NH,), jnp.float32)),
        mesh=mesh,
        compiler_params=pltpu.CompilerParams(needs_layout_passes=False, use_tc_tiling_on_sc=False),
        scratch_types=[
            pltpu.VMEM((NPAD * NH,), jnp.float32),
            pltpu.VMEM((NPAD * NH,), jnp.float32),
            pltpu.VMEM((C1,), jnp.int32),
            pltpu.VMEM((C1,), jnp.int32),
            pltpu.VMEM((C1, 16), jnp.float32),
            pltpu.VMEM((C1 * NH,), jnp.float32),
            pltpu.VMEM((ROWS_T, 16), jnp.float32),
            pltpu.VMEM_SHARED((NPAD, 16), jnp.float32),
        ],
    )
    def k(s_hbm, d_hbm, asrc_hbm, adst_hbm, denom_hbm, p_hbm,
          asrc_t, adst_t, sidx, didx, p16, p4, zbuf, denom_sh):
        c = lax.axis_index("c")
        s = lax.axis_index("s")
        tid = c * NS + s
        pltpu.sync_copy(asrc_hbm, asrc_t)
        pltpu.sync_copy(adst_hbm, adst_t)
        zero16 = jnp.zeros((16,), jnp.float32)

        def zrow_z(i, _):
            zbuf[i, :] = zero16
            return 0

        lax.fori_loop(0, ROWS_T, zrow_z, 0)

        def zrow_p(i, _):
            p16[i, :] = zero16
            return 0

        lax.fori_loop(0, C1, zrow_p, 0)
        pltpu.sync_copy(zbuf, denom_sh.at[pl.ds(s * ROWS_T, ROWS_T)])
        plsc.subcore_barrier()

        def chunk(kk, _):
            off = tid * T_EDGE + kk * C1
            pltpu.sync_copy(s_hbm.at[pl.ds(off, C1)], sidx)
            pltpu.sync_copy(d_hbm.at[pl.ds(off, C1)], didx)

            def group(j, _):
                rows = j * 16 + _I16()
                sv = plsc.load_gather(sidx, [rows])
                dv = plsc.load_gather(didx, [rows])
                for h in range(NH):
                    a1 = plsc.load_gather(asrc_t, [sv * NH + h])
                    a2 = plsc.load_gather(adst_t, [dv * NH + h])
                    al = a1 + a2
                    al = jnp.where(al >= 0, al, 0.2 * al)
                    pv = jnp.exp(al)
                    plsc.store_scatter(
                        p16, [rows, jnp.full((16,), h, jnp.int32)], pv)
                    plsc.store_scatter(p4, [rows * NH + h], pv)
                return 0

            lax.fori_loop(0, C1 // 16, group, 0)
            pltpu.sync_copy(p16, denom_sh.at[didx], add=True)
            pltpu.sync_copy(p4, p_hbm.at[pl.ds(off * NH, C1 * NH)])
            return 0

        lax.fori_loop(0, T_EDGE // C1, chunk, 0)
        plsc.subcore_barrier()
        pltpu.sync_copy(denom_sh.at[pl.ds(s * ROWS_T, ROWS_T)], zbuf)
        pltpu.sync_copy(zbuf,
                        denom_hbm.at[pl.ds(c * NPAD + s * ROWS_T, ROWS_T)])

    return k(s_ext, d_ext, asrc_flat, adst_flat)


def _sc_pass2(s_ext, d_ext, w_flat, x):
    mesh = plsc.VectorSubcoreMesh(core_axis_name="c", subcore_axis_name="s", num_cores=NC, num_subcores=NS)

    @functools.partial(
        pl.kernel,
        out_type=jax.ShapeDtypeStruct((NC * NPAD, D), jnp.float32),
        mesh=mesh,
        compiler_params=pltpu.CompilerParams(needs_layout_passes=False, use_tc_tiling_on_sc=False),
        scratch_types=[
            pltpu.VMEM((C2,), jnp.int32),
            pltpu.VMEM((C2,), jnp.int32),
            pltpu.VMEM((C2 * NH,), jnp.float32),
            pltpu.VMEM((C2, NH * D), jnp.float32),
            pltpu.VMEM((C2, D), jnp.float32),
            pltpu.VMEM_SHARED((NPAD, D), jnp.float32),
            pltpu.SemaphoreType.DMA,
        ],
    )
    def k(s_hbm, d_hbm, w_hbm, x_hbm, out_hbm,
          sidx, didx, wbuf, xbuf, mbuf, out_sh, sem):
        c = lax.axis_index("c")
        s = lax.axis_index("s")
        tid = c * NS + s
        zero16 = jnp.zeros((16,), jnp.float32)

        def zrow_m(i, _):
            for v in range(D // 16):
                mbuf[i, pl.ds(v * 16, 16)] = zero16
            return 0

        lax.fori_loop(0, C2, zrow_m, 0)
        for i in range(10):
            r0 = s * ROWS_T + i * C2
            sz = C2 if i < 9 else ROWS_T - 9 * C2
            pltpu.sync_copy(mbuf.at[pl.ds(0, sz)],
                            out_sh.at[pl.ds(r0, sz)])
        plsc.subcore_barrier()

        def chunk(kk, _):
            off = tid * T_EDGE + kk * C2
            pltpu.sync_copy(s_hbm.at[pl.ds(off, C2)], sidx)
            pltpu.sync_copy(d_hbm.at[pl.ds(off, C2)], didx)
            pltpu.sync_copy(w_hbm.at[pl.ds(off * NH, C2 * NH)], wbuf)
            pltpu.async_copy(x_hbm.at[sidx], xbuf, sem).wait()

            def edge(e, _):
                wv = plsc.load_gather(
                    wbuf, [e * NH + jnp.minimum(_I16(), NH - 1)])
                w0 = wv[0]
                w1 = wv[1]
                w2 = wv[2]
                w3 = wv[3]
                for v in range(D // 16):
                    acc = w0 * xbuf[e, pl.ds(v * 16, 16)]
                    acc += w1 * xbuf[e, pl.ds(D + v * 16, 16)]
                    acc += w2 * xbuf[e, pl.ds(2 * D + v * 16, 16)]
                    acc += w3 * xbuf[e, pl.ds(3 * D + v * 16, 16)]
                    mbuf[e, pl.ds(v * 16, 16)] = acc
                return 0

            lax.fori_loop(0, C2, edge, 0)
            pltpu.sync_copy(mbuf, out_sh.at[didx], add=True)
            return 0

        lax.fori_loop(0, T_EDGE // C2, chunk, 0)
        plsc.subcore_barrier()
        for i in range(10):
            r0 = s * ROWS_T + i * C2
            sz = C2 if i < 9 else ROWS_T - 9 * C2
            pltpu.sync_copy(out_sh.at[pl.ds(r0, sz)], mbuf.at[pl.ds(0, sz)])
            pltpu.sync_copy(mbuf.at[pl.ds(0, sz)],
                            out_hbm.at[pl.ds(c * NPAD + r0, sz)])

    return k(s_ext, d_ext, w_flat, x)


def _sc_weights(d_ext, p_flat, winv_flat):
    mesh = plsc.VectorSubcoreMesh(core_axis_name="c", subcore_axis_name="s", num_cores=NC, num_subcores=NS)

    @functools.partial(
        pl.kernel,
        out_type=jax.ShapeDtypeStruct((EPAD * NH,), jnp.float32),
        mesh=mesh,
        compiler_params=pltpu.CompilerParams(needs_layout_passes=False, use_tc_tiling_on_sc=False),
        scratch_types=[
            pltpu.VMEM((NPAD * NH,), jnp.float32),
            pltpu.VMEM((C1,), jnp.int32),
            pltpu.VMEM((C1 * NH,), jnp.float32),
            pltpu.VMEM((C1 * NH,), jnp.float32),
        ],
    )
    def k(d_hbm, p_hbm, winv_hbm, w_hbm, winv_t, didx, pbuf, wbuf):
        c = lax.axis_index("c")
        s = lax.axis_index("s")
        tid = c * NS + s
        pltpu.sync_copy(winv_hbm, winv_t)

        def chunk(kk, _):
            off = tid * T_EDGE + kk * C1
            pltpu.sync_copy(d_hbm.at[pl.ds(off, C1)], didx)
            pltpu.sync_copy(p_hbm.at[pl.ds(off * NH, C1 * NH)], pbuf)

            def group(j, _):
                rows = j * 16 + _I16()
                dv = plsc.load_gather(didx, [rows])
                for h in range(NH):
                    wv = plsc.load_gather(winv_t, [dv * NH + h])
                    pv = plsc.load_gather(pbuf, [rows * NH + h])
                    plsc.store_scatter(wbuf, [rows * NH + h], wv * pv)
                return 0

            lax.fori_loop(0, C1 // 16, group, 0)
            pltpu.sync_copy(wbuf, w_hbm.at[pl.ds(off * NH, C1 * NH)])
            return 0

        lax.fori_loop(0, T_EDGE // C1, chunk, 0)

    return k(d_ext, p_flat, winv_flat)


def _edge_phase(x, a_src_p, a_dst_p, s_ext, d_ext):
    denom, p = _sc_pass1(s_ext, d_ext,
                         a_src_p.reshape(-1), a_dst_p.reshape(-1))
    winv = _winv(denom[:NPAD].reshape(-1, 128),
                 denom[NPAD:].reshape(-1, 128))
    winv_flat = winv.reshape(NPAD, 16)[:, :NH].reshape(-1)
    w = _sc_weights(d_ext, p, winv_flat)
    out2 = _sc_pass2(s_ext, d_ext, w, x)
    return out2[:NPAD][:N], out2[NPAD:][:N]


def kernel(x_item, x_seq, edge_index, W_item, W_seq,
           lin0, att_src0, att_dst0, bias0,
           lin1, att_src1, att_dst1, bias1):
    f32 = jnp.float32
    src = edge_index[0].astype(jnp.int32)
    dst = edge_index[1].astype(jnp.int32)
    loop = jnp.arange(N, dtype=jnp.int32)
    npad_e = EPAD - E_EXT
    s_ext = jnp.concatenate([src, loop, jnp.zeros((npad_e,), jnp.int32)])
    d_ext = jnp.concatenate([dst, loop, jnp.full((npad_e,), N, jnp.int32)])

    def make_ab(att_s, att_d):
        ab = jnp.zeros((NH * D, D), f32)
        for h in range(NH):
            ab = ab.at[h * D:(h + 1) * D, h].set(att_s[0, h])
            ab = ab.at[h * D:(h + 1) * D, NH + h].set(att_d[0, h])
        return ab

    AB0 = make_ab(att_src0, att_dst0)
    AB1 = make_ab(att_src1, att_dst1)

    h_i = _h_item(x_item, W_item.T)
    h_s = _h_seq(x_seq, W_seq.T)
    H = jnp.concatenate([h_i, h_s], axis=0)

    x0, a0 = _proj(H, lin0.T, AB0)
    a_src_p = jnp.zeros((NPAD, NH), f32).at[:N].set(a0[:, :NH])
    a_dst_p = jnp.zeros((NPAD, NH), f32).at[:N].set(a0[:, NH:2 * NH])
    G0a, G0b = _edge_phase(x0, a_src_p, a_dst_p, s_ext, d_ext)
    R0, s1_0, s2_0 = _stats(G0a, G0b, bias0.reshape(1, D))

    x1, a1 = _normproj(R0, s1_0, s2_0, lin1.T, AB1)
    a_src_p = jnp.zeros((NPAD, NH), f32).at[:N].set(a1[:, :NH])
    a_dst_p = jnp.zeros((NPAD, NH), f32).at[:N].set(a1[:, NH:2 * NH])
    G1a, G1b = _edge_phase(x1, a_src_p, a_dst_p, s_ext, d_ext)
    R1, s1_1, s2_1 = _stats(G1a, G1b, bias1.reshape(1, D))
    Hf = _norm(R1, s1_1, s2_1)

    return (Hf[:N_ITEM], Hf[N_ITEM:])
